# baseline (speedup 1.0000x reference)
import jax
import jax.numpy as jnp
from jax import lax
from jax.experimental import pallas as pl
from jax.experimental.pallas import tpu as pltpu

N_DOMAIN = 8
EPS = 1e-05
ROWS = 16384
COLS = 1024
BR = 1024
NB = ROWS // BR
COLH = 512
NH = COLS // COLH
NWB = 8


def _onehot_t(y_ref, i):
    yv = y_ref[i]
    ids = lax.broadcasted_iota(jnp.int32, (N_DOMAIN, BR), 0)
    return (ids == yv).astype(jnp.float32)


def _kernel(y_ref, g_ref, b_ref, x_any, out_any,
            xbuf, ob0, ob1, ob2, ob3, ob4, ob5, ob6, ob7,
            sums, sumsq, cnt, atab, btab, rs, ws):
    h = pl.program_id(0)
    p = pl.program_id(1)
    obs = [ob0, ob1, ob2, ob3, ob4, ob5, ob6, ob7]

    def rds(blk, hh):
        half = BR // 2
        return [pltpu.make_async_copy(
            x_any.at[pl.ds(blk * BR + k * half, half),
                     pl.ds(hh * COLH, COLH)],
            xbuf.at[pl.ds(blk * BR + k * half, half), :],
            rs.at[2 * blk + k]) for k in (0, 1)]

    def rd_start(blk, hh):
        for c in rds(blk, hh):
            c.start()

    def rd_wait(blk, hh):
        for c in rds(blk, hh):
            c.wait()

    def wr(blk, obuf):
        return pltpu.make_async_copy(
            obuf, out_any.at[pl.ds(blk * BR, BR), pl.ds(h * COLH, COLH)],
            ws.at[blk % NWB])

    @pl.when(p == 0)
    def _phase0():
        @pl.when(h == 0)
        def _prime():
            for j in range(NB):
                rd_start(j, h)

        sums[...] = jnp.zeros_like(sums)
        sumsq[...] = jnp.zeros_like(sumsq)

        @pl.when(h == 0)
        def _zc():
            cnt[...] = jnp.zeros_like(cnt)

        for i in range(NB):
            rd_wait(i, h)
            xb = xbuf[pl.ds(i * BR, BR), :]
            oh = _onehot_t(y_ref, i)
            sums[...] += lax.dot_general(
                oh, xb, (((1,), (0,)), ((), ())),
                preferred_element_type=jnp.float32)
            sumsq[...] += lax.dot_general(
                oh, xb * xb, (((1,), (0,)), ((), ())),
                preferred_element_type=jnp.float32)

            @pl.when(h == 0)
            def _count():
                cnt[...] += jnp.broadcast_to(
                    jnp.sum(oh, axis=1, keepdims=True), cnt.shape)

    @pl.when(p == 1)
    def _phase1():
        c = cnt[:, :1]
        denom = jnp.maximum(c, 1.0)
        mean = sums[...] / denom
        var = jnp.maximum(sumsq[...] / denom - mean * mean, 0.0)
        gh = g_ref[:, pl.ds(h * COLH, COLH)]
        bh = b_ref[:, pl.ds(h * COLH, COLH)]
        scale = gh * lax.rsqrt(var + EPS)
        multi = c > 1.0
        atab[...] = jnp.where(multi, scale, 1.0)
        btab[...] = jnp.where(multi, bh - mean * scale, 0.0)

        for i in range(NB):
            obuf = obs[i % NWB]
            if i >= NWB:
                wr(i - NWB, obuf).wait()
            oh = _onehot_t(y_ref, i)
            row_a = lax.dot_general(
                oh, atab[...], (((0,), (0,)), ((), ())),
                preferred_element_type=jnp.float32)
            row_b = lax.dot_general(
                oh, btab[...], (((0,), (0,)), ((), ())),
                preferred_element_type=jnp.float32)
            obuf[...] = xbuf[pl.ds(i * BR, BR), :] * row_a + row_b
            wr(i, obuf).start()

        @pl.when(h + 1 < NH)
        def _prefetch():
            for j in range(NB):
                rd_start(j, h + 1)

        for j in range(NB - NWB, NB):
            wr(j, obs[j % NWB]).wait()


@jax.jit
def kernel(x, y, gamma, beta):
    y3 = y.astype(jnp.int32).reshape(NB, 1, BR)
    out = pl.pallas_call(
        _kernel,
        grid=(NH, 2),
        in_specs=[
            pl.BlockSpec((NB, 1, BR), lambda h, p: (0, 0, 0)),
            pl.BlockSpec((1, COLS), lambda h, p: (0, 0)),
            pl.BlockSpec((1, COLS), lambda h, p: (0, 0)),
            pl.BlockSpec(memory_space=pl.ANY),
        ],
        out_specs=pl.BlockSpec(memory_space=pl.ANY),
        out_shape=jax.ShapeDtypeStruct((ROWS, COLS), jnp.float32),
        scratch_shapes=[
            pltpu.VMEM((ROWS, COLH), jnp.float32),
        ] + [pltpu.VMEM((BR, COLH), jnp.float32) for _ in range(8)] + [
            pltpu.VMEM((N_DOMAIN, COLH), jnp.float32),
            pltpu.VMEM((N_DOMAIN, COLH), jnp.float32),
            pltpu.VMEM((N_DOMAIN, 128), jnp.float32),
            pltpu.VMEM((N_DOMAIN, COLH), jnp.float32),
            pltpu.VMEM((N_DOMAIN, COLH), jnp.float32),
            pltpu.SemaphoreType.DMA((2 * NB,)),
            pltpu.SemaphoreType.DMA((NWB,)),
        ],
    )(y3, gamma, beta, x)
    return out


# speedup vs baseline: 1.0094x; 1.0094x over previous
import jax
import jax.numpy as jnp
from jax import lax
from jax.experimental import pallas as pl
from jax.experimental.pallas import tpu as pltpu

N_DOMAIN = 8
EPS = 1e-05
ROWS = 16384
COLS = 1024
BR = 1024
NB = ROWS // BR
COLH = 512
NH = COLS // COLH
NWB = 12


def _onehot_t(y_ref, i):
    yv = y_ref[i]
    ids = lax.broadcasted_iota(jnp.int32, (N_DOMAIN, BR), 0)
    return (ids == yv).astype(jnp.float32)


def _kernel(y_ref, g_ref, b_ref, x_any, out_any,
            xbuf, ob0, ob1, ob2, ob3, ob4, ob5, ob6, ob7, ob8, ob9, ob10,
            ob11, sums, sumsq, cnt, atab, btab, rs, ws):
    h = pl.program_id(0)
    p = pl.program_id(1)
    obs = [ob0, ob1, ob2, ob3, ob4, ob5, ob6, ob7, ob8, ob9, ob10, ob11]

    def rd(blk, hh):
        return pltpu.make_async_copy(
            x_any.at[pl.ds(blk * BR, BR), pl.ds(hh * COLH, COLH)],
            xbuf.at[pl.ds(blk * BR, BR), :], rs.at[blk])

    def wr(blk, obuf):
        return pltpu.make_async_copy(
            obuf, out_any.at[pl.ds(blk * BR, BR), pl.ds(h * COLH, COLH)],
            ws.at[blk % NWB])

    @pl.when(p == 0)
    def _phase0():
        @pl.when(h == 0)
        def _prime():
            for j in range(NB):
                rd(j, h).start()

        sums[...] = jnp.zeros_like(sums)
        sumsq[...] = jnp.zeros_like(sumsq)

        @pl.when(h == 0)
        def _zc():
            cnt[...] = jnp.zeros_like(cnt)

        for i in range(NB):
            rd(i, h).wait()
            xb = xbuf[pl.ds(i * BR, BR), :]
            oh = _onehot_t(y_ref, i)
            sums[...] += lax.dot_general(
                oh, xb, (((1,), (0,)), ((), ())),
                preferred_element_type=jnp.float32)
            sumsq[...] += lax.dot_general(
                oh, xb * xb, (((1,), (0,)), ((), ())),
                preferred_element_type=jnp.float32)

            @pl.when(h == 0)
            def _count():
                cnt[...] += jnp.broadcast_to(
                    jnp.sum(oh, axis=1, keepdims=True), cnt.shape)

    @pl.when(p == 1)
    def _phase1():
        c = cnt[:, :1]
        denom = jnp.maximum(c, 1.0)
        mean = sums[...] / denom
        var = jnp.maximum(sumsq[...] / denom - mean * mean, 0.0)
        gh = g_ref[:, pl.ds(h * COLH, COLH)]
        bh = b_ref[:, pl.ds(h * COLH, COLH)]
        scale = gh * lax.rsqrt(var + EPS)
        multi = c > 1.0
        atab[...] = jnp.where(multi, scale, 1.0)
        btab[...] = jnp.where(multi, bh - mean * scale, 0.0)

        for i in range(NB):
            obuf = obs[i % NWB]
            if i >= NWB:
                wr(i - NWB, obuf).wait()
            oh = _onehot_t(y_ref, i)
            row_a = lax.dot_general(
                oh, atab[...], (((0,), (0,)), ((), ())),
                preferred_element_type=jnp.float32)
            row_b = lax.dot_general(
                oh, btab[...], (((0,), (0,)), ((), ())),
                preferred_element_type=jnp.float32)
            obuf[...] = xbuf[pl.ds(i * BR, BR), :] * row_a + row_b
            wr(i, obuf).start()

        @pl.when(h + 1 < NH)
        def _prefetch():
            for j in range(NB):
                rd(j, h + 1).start()

        for j in range(NB - NWB, NB):
            wr(j, obs[j % NWB]).wait()


@jax.jit
def kernel(x, y, gamma, beta):
    y3 = y.astype(jnp.int32).reshape(NB, 1, BR)
    out = pl.pallas_call(
        _kernel,
        grid=(NH, 2),
        in_specs=[
            pl.BlockSpec((NB, 1, BR), lambda h, p: (0, 0, 0)),
            pl.BlockSpec((1, COLS), lambda h, p: (0, 0)),
            pl.BlockSpec((1, COLS), lambda h, p: (0, 0)),
            pl.BlockSpec(memory_space=pl.ANY),
        ],
        out_specs=pl.BlockSpec(memory_space=pl.ANY),
        out_shape=jax.ShapeDtypeStruct((ROWS, COLS), jnp.float32),
        scratch_shapes=[
            pltpu.VMEM((ROWS, COLH), jnp.float32),
        ] + [pltpu.VMEM((BR, COLH), jnp.float32) for _ in range(12)] + [
            pltpu.VMEM((N_DOMAIN, COLH), jnp.float32),
            pltpu.VMEM((N_DOMAIN, COLH), jnp.float32),
            pltpu.VMEM((N_DOMAIN, 128), jnp.float32),
            pltpu.VMEM((N_DOMAIN, COLH), jnp.float32),
            pltpu.VMEM((N_DOMAIN, COLH), jnp.float32),
            pltpu.SemaphoreType.DMA((NB,)),
            pltpu.SemaphoreType.DMA((NWB,)),
        ],
    )(y3, gamma, beta, x)
    return out
